# Initial kernel scaffold; baseline (speedup 1.0000x reference)
import jax
import jax.numpy as jnp
from jax.experimental import pallas as pl


def kernel(atom_features, bond_info, params):
    raise NotImplementedError("write your pallas kernel here")


# speedup vs baseline: 4.5227x; 4.5227x over previous
import functools

import jax
import jax.numpy as jnp
from jax import lax
from jax.experimental import pallas as pl
from jax.experimental.pallas import tpu as pltpu
from jax.experimental.pallas import tpu_sc as plsc

_N = 10000
_E = 320000
_NBT = 4
_N4 = _NBT * _N
_N4P = _N4 + 16
_NT = 16
_EB = 128
_ET = 20096
_NBATCH = _ET // _EB
_EP = _NT * _ET
_RT0 = 2496
_RTL = _N4 - (_NT - 1) * _RT0
_CW = 16
_ROWB = 400
_NRB = _N // _ROWB


_XR0 = 624
_XRL = _N - (_NT - 1) * _XR0


def _make_msg_kernel(C):
    npc = C // 2
    mesh = plsc.VectorSubcoreMesh(core_axis_name="c", subcore_axis_name="s",
                                  num_cores=2, num_subcores=_NT)

    @functools.partial(
        pl.kernel,
        out_type=jax.ShapeDtypeStruct((C * _N4, _CW), jnp.float32),
        mesh=mesh,
        compiler_params=pltpu.CompilerParams(use_tc_tiling_on_sc=False),
        scratch_types=[
            pltpu.VMEM_SHARED((_N4P, _CW), jnp.float32),
            pltpu.VMEM_SHARED((_N, _CW), jnp.float32),
            pltpu.VMEM((_NBATCH, _EB), jnp.int32),
            pltpu.VMEM((_NBATCH, _EB), jnp.int32),
            pltpu.VMEM((_NBATCH, _EB), jnp.int32),
            pltpu.VMEM((2, _EB, _CW), jnp.float32),
            pltpu.SemaphoreType.DMA,
            pltpu.SemaphoreType.DMA,
        ],
    )
    def msg_kernel(xcm, beg_h, end_h, bt_h, zeros_h, out_h,
                   msg_sh, x_sp, beg_v, dst_v, bt_v, rows_v, sem0, sem1):
        cid = lax.axis_index("c")
        sid = lax.axis_index("s")

        pltpu.sync_copy(beg_h.at[sid], beg_v)
        pltpu.sync_copy(end_h.at[sid], dst_v)
        pltpu.sync_copy(bt_h.at[sid], bt_v)

        @pl.loop(0, _NBATCH)
        def _(b):
            for i in range(_EB // 16):
                s = pl.ds(i * 16, 16)
                dst_v[b, s] = dst_v[b, s] * _NBT + bt_v[b, s]

        for j in range(npc):
            c = cid * npc + j

            @pl.when(sid < _NT - 1)
            def _():
                pltpu.sync_copy(
                    xcm.at[pl.ds(c * _N + sid * _XR0, _XR0)],
                    x_sp.at[pl.ds(sid * _XR0, _XR0)])

            @pl.when(sid == _NT - 1)
            def _():
                pltpu.sync_copy(
                    xcm.at[pl.ds(c * _N + (_NT - 1) * _XR0, _XRL)],
                    x_sp.at[pl.ds((_NT - 1) * _XR0, _XRL)])

            @pl.when(sid < _NT - 1)
            def _():
                pltpu.sync_copy(zeros_h.at[pl.ds(0, _RT0)],
                                msg_sh.at[pl.ds(sid * _RT0, _RT0)])

            @pl.when(sid == _NT - 1)
            def _():
                pltpu.sync_copy(zeros_h,
                                msg_sh.at[pl.ds((_NT - 1) * _RT0, _RTL)])

            plsc.subcore_barrier()

            pltpu.async_copy(x_sp.at[beg_v.at[0]], rows_v.at[0], sem0)

            @pl.loop(0, _NBATCH // 2)
            def _(k):
                b0 = k * 2
                pltpu.make_async_copy(
                    x_sp.at[beg_v.at[b0]], rows_v.at[0], sem0).wait()
                pltpu.async_copy(
                    x_sp.at[beg_v.at[b0 + 1]], rows_v.at[1], sem1)
                pltpu.sync_copy(
                    rows_v.at[0], msg_sh.at[dst_v.at[b0]], add=True)
                pltpu.make_async_copy(
                    x_sp.at[beg_v.at[b0 + 1]], rows_v.at[1], sem1).wait()
                pltpu.async_copy(
                    x_sp.at[beg_v.at[b0 + 2]], rows_v.at[0], sem0)
                pltpu.sync_copy(
                    rows_v.at[1], msg_sh.at[dst_v.at[b0 + 1]], add=True)

            bl = _NBATCH - 1
            pltpu.make_async_copy(
                x_sp.at[beg_v.at[bl]], rows_v.at[0], sem0).wait()
            pltpu.sync_copy(rows_v.at[0], msg_sh.at[dst_v.at[bl]], add=True)

            plsc.subcore_barrier()

            @pl.when(sid < _NT - 1)
            def _():
                pltpu.sync_copy(
                    msg_sh.at[pl.ds(sid * _RT0, _RT0)],
                    out_h.at[pl.ds(c * _N4 + sid * _RT0, _RT0)])

            @pl.when(sid == _NT - 1)
            def _():
                pltpu.sync_copy(
                    msg_sh.at[pl.ds((_NT - 1) * _RT0, _RTL)],
                    out_h.at[pl.ds(c * _N4 + (_NT - 1) * _RT0, _RTL)])

            if j + 1 < npc:
                plsc.subcore_barrier()

    return msg_kernel


_make_msg_kernel = functools.lru_cache(maxsize=None)(_make_msg_kernel)


def _stats(x3):
    G, _, K = x3.shape

    def body(x_ref, o_ref, acc):
        i = pl.program_id(1)

        @pl.when(i == 0)
        def _():
            acc[...] = jnp.zeros_like(acc)

        xb = x_ref[0]
        acc[0:1, :] += jnp.sum(xb, axis=0, keepdims=True)
        acc[1:2, :] += jnp.sum(xb * xb, axis=0, keepdims=True)

        @pl.when(i == _NRB - 1)
        def _():
            o_ref[0] = acc[...]

    return pl.pallas_call(
        body,
        grid=(G, _NRB),
        in_specs=[pl.BlockSpec((1, _ROWB, K), lambda g, i: (g, i, 0))],
        out_specs=pl.BlockSpec((1, 2, K), lambda g, i: (g, 0, 0)),
        out_shape=jax.ShapeDtypeStruct((G, 2, K), jnp.float32),
        scratch_shapes=[pltpu.VMEM((2, K), jnp.float32)],
    )(x3)


def _apply(blocks, ss_list, w_list, bias, dout):
    n_in = len(blocks)
    n_sub = len(w_list)

    def body(*refs):
        bl_refs = refs[:n_in]
        ss_refs = refs[n_in:n_in + n_sub]
        w_refs = refs[n_in + n_sub:n_in + 2 * n_sub]
        bias_ref = refs[n_in + 2 * n_sub]
        out_ref, st_ref, acc = refs[n_in + 2 * n_sub + 1:]
        i = pl.program_id(0)

        @pl.when(i == 0)
        def _():
            acc[...] = jnp.zeros_like(acc)

        subs = []
        for r, arr in zip(bl_refs, blocks):
            if arr.ndim == 3:
                for cc in range(arr.shape[0]):
                    subs.append(r[cc])
            else:
                subs.append(r[...])

        tot = None
        for xb, ssr, wr in zip(subs, ss_refs, w_refs):
            s = ssr[...]
            xn = xb * s[0:1, :] + s[1:2, :]
            e = jnp.where(xn > 0, xn, jnp.exp(xn) - 1.0)
            d = jnp.dot(e, wr[...], precision=lax.Precision.HIGHEST,
                        preferred_element_type=jnp.float32)
            tot = d if tot is None else tot + d
        tot = tot + bias_ref[...]
        out_ref[...] = tot
        acc[0:1, :] += jnp.sum(tot, axis=0, keepdims=True)
        acc[1:2, :] += jnp.sum(tot * tot, axis=0, keepdims=True)

        @pl.when(i == _NRB - 1)
        def _():
            st_ref[...] = acc[...]

    in_specs = []
    for arr in blocks:
        if arr.ndim == 3:
            Cc, _, Kc = arr.shape
            in_specs.append(
                pl.BlockSpec((Cc, _ROWB, Kc), lambda i: (0, i, 0)))
        else:
            K = arr.shape[1]
            in_specs.append(pl.BlockSpec((_ROWB, K), lambda i: (i, 0)))
    for s in ss_list:
        in_specs.append(pl.BlockSpec(s.shape, lambda i: (0, 0)))
    for w in w_list:
        in_specs.append(pl.BlockSpec(w.shape, lambda i: (0, 0)))
    in_specs.append(pl.BlockSpec((1, dout), lambda i: (0, 0)))

    out, st = pl.pallas_call(
        body,
        grid=(_NRB,),
        in_specs=in_specs,
        out_specs=[
            pl.BlockSpec((_ROWB, dout), lambda i: (i, 0)),
            pl.BlockSpec((2, dout), lambda i: (0, 0)),
        ],
        out_shape=[
            jax.ShapeDtypeStruct((_N, dout), jnp.float32),
            jax.ShapeDtypeStruct((2, dout), jnp.float32),
        ],
        scratch_shapes=[pltpu.VMEM((2, dout), jnp.float32)],
    )(*blocks, *ss_list, *w_list, bias)
    return out, st


def _ss(stats, gamma, beta):
    mean = stats[0] / _N
    var = stats[1] / _N - mean * mean
    scale = gamma * lax.rsqrt(var + 1e-5)
    return jnp.stack([scale, beta - mean * scale])


def _mol_conv_layer(x, xstats, edges, zeros, p, F):
    C = F // _CW
    KM = _NBT * _CW
    beg3, end3, bt3 = edges
    xcm = x.reshape(_N, C, _CW).transpose(1, 0, 2).reshape(C * _N, _CW)
    msg = _make_msg_kernel(C)(xcm, beg3, end3, bt3, zeros)
    msg3 = msg.reshape(C, _N, KM)
    mstats = _stats(msg3)

    W, gamma, beta = p["W"], p["gamma"], p["beta"]
    dout = W.shape[0]
    gm = gamma[F:].reshape(_NBT, C, _CW)
    bm = beta[F:].reshape(_NBT, C, _CW)
    Wm = W[:, F:].reshape(dout, _NBT, C, _CW)

    ss_list = [_ss(xstats, gamma[:F], beta[:F])]
    w_list = [W[:, :F].T]
    for c in range(C):
        ss_list.append(_ss(mstats[c], gm[:, c, :].reshape(KM),
                           bm[:, c, :].reshape(KM)))
        w_list.append(Wm[:, :, c, :].reshape(dout, KM).T)
    return _apply([x, msg3], ss_list, w_list, p["b"].reshape(1, dout), dout)


def _dense_apply(feats, p, dout):
    W, gamma, beta = p["W"], p["gamma"], p["beta"]
    blocks, ss_list, w_list = [], [], []
    off = 0
    for arr, st in feats:
        K = arr.shape[1]
        blocks.append(arr)
        ss_list.append(_ss(st, gamma[off:off + K], beta[off:off + K]))
        w_list.append(W[:, off:off + K].T)
        off += K
    return _apply(blocks, ss_list, w_list, p["b"].reshape(1, dout), dout)


def kernel(atom_features, bond_info, params):
    beg = bond_info[:, 0]
    end = bond_info[:, 1]
    bt = bond_info[:, 2]
    pad = _EP - _E
    beg3 = jnp.concatenate(
        [beg, jnp.zeros((pad,), jnp.int32)]).reshape(_NT, _NBATCH, _EB)
    end3 = jnp.concatenate(
        [end, jnp.full((pad,), _N, jnp.int32)]).reshape(_NT, _NBATCH, _EB)
    bt3 = jnp.concatenate(
        [bt, jnp.zeros((pad,), jnp.int32)]).reshape(_NT, _NBATCH, _EB)
    edges = (beg3, end3, bt3)
    zeros = jnp.zeros((_RTL, _CW), jnp.float32)

    x = atom_features
    xstats = _stats(x.reshape(1, _N, 128))[0]
    for p in params["causal"]:
        x, xstats = _mol_conv_layer(x, xstats, edges, zeros, p, 128)

    feats = [(x, xstats)]
    for lp in params["dense"]:
        bn, bnstats = _dense_apply(feats, lp["bottleneck"], 64)
        k, kstats = _mol_conv_layer(bn, bnstats, edges, zeros, lp["conv"], 64)
        feats.append((k, kstats))

    out, _ = _dense_apply(feats, params["output"], 128)
    return out


# speedup vs baseline: 4.5340x; 1.0025x over previous
import functools

import jax
import jax.numpy as jnp
from jax import lax
from jax.experimental import pallas as pl
from jax.experimental.pallas import tpu as pltpu
from jax.experimental.pallas import tpu_sc as plsc

_N = 10000
_E = 320000
_NBT = 4
_N4 = _NBT * _N
_N4P = _N4 + 16
_NT = 16
_EB = 128
_ET = 20096
_NBATCH = _ET // _EB
_EP = _NT * _ET
_RT0 = 2496
_RTL = _N4 - (_NT - 1) * _RT0
_CW = 32
_ROWB = 400
_NRB = _N // _ROWB


_XR0 = 624
_XRL = _N - (_NT - 1) * _XR0


def _make_msg_kernel(C):
    npc = C // 2
    mesh = plsc.VectorSubcoreMesh(core_axis_name="c", subcore_axis_name="s",
                                  num_cores=2, num_subcores=_NT)

    @functools.partial(
        pl.kernel,
        out_type=jax.ShapeDtypeStruct((C * _N4, _CW), jnp.float32),
        mesh=mesh,
        compiler_params=pltpu.CompilerParams(use_tc_tiling_on_sc=False),
        scratch_types=[
            pltpu.VMEM_SHARED((_N4P, _CW), jnp.float32),
            pltpu.VMEM((_NBATCH, _EB), jnp.int32),
            pltpu.VMEM((_NBATCH, _EB), jnp.int32),
            pltpu.VMEM((2, _EB, _CW), jnp.float32),
            pltpu.SemaphoreType.DMA,
            pltpu.SemaphoreType.DMA,
        ],
    )
    def msg_kernel(xcm, beg_h, end_h, bt_h, zeros_h, out_h,
                   msg_sh, beg_v, dst_v, rows_v, sem0, sem1):
        cid = lax.axis_index("c")
        sid = lax.axis_index("s")

        pltpu.sync_copy(end_h.at[sid], dst_v)
        pltpu.sync_copy(bt_h.at[sid], beg_v)

        @pl.loop(0, _NBATCH)
        def _(b):
            for i in range(_EB // 16):
                s = pl.ds(i * 16, 16)
                dst_v[b, s] = dst_v[b, s] * _NBT + beg_v[b, s]

        pltpu.sync_copy(beg_h.at[sid], beg_v)

        for j in range(npc):
            c = cid * npc + j

            xc = xcm.at[pl.ds(c * _N, _N)]

            @pl.when(sid < _NT - 1)
            def _():
                pltpu.sync_copy(zeros_h.at[pl.ds(0, _RT0)],
                                msg_sh.at[pl.ds(sid * _RT0, _RT0)])

            @pl.when(sid == _NT - 1)
            def _():
                pltpu.sync_copy(zeros_h,
                                msg_sh.at[pl.ds((_NT - 1) * _RT0, _RTL)])

            plsc.subcore_barrier()

            pltpu.async_copy(xc.at[beg_v.at[0]], rows_v.at[0], sem0)

            @pl.loop(0, _NBATCH // 2)
            def _(k):
                b0 = k * 2
                pltpu.make_async_copy(
                    xc.at[beg_v.at[b0]], rows_v.at[0], sem0).wait()
                pltpu.async_copy(
                    xc.at[beg_v.at[b0 + 1]], rows_v.at[1], sem1)
                pltpu.sync_copy(
                    rows_v.at[0], msg_sh.at[dst_v.at[b0]], add=True)
                pltpu.make_async_copy(
                    xc.at[beg_v.at[b0 + 1]], rows_v.at[1], sem1).wait()
                pltpu.async_copy(
                    xc.at[beg_v.at[b0 + 2]], rows_v.at[0], sem0)
                pltpu.sync_copy(
                    rows_v.at[1], msg_sh.at[dst_v.at[b0 + 1]], add=True)

            bl = _NBATCH - 1
            pltpu.make_async_copy(
                xc.at[beg_v.at[bl]], rows_v.at[0], sem0).wait()
            pltpu.sync_copy(rows_v.at[0], msg_sh.at[dst_v.at[bl]], add=True)

            plsc.subcore_barrier()

            @pl.when(sid < _NT - 1)
            def _():
                pltpu.sync_copy(
                    msg_sh.at[pl.ds(sid * _RT0, _RT0)],
                    out_h.at[pl.ds(c * _N4 + sid * _RT0, _RT0)])

            @pl.when(sid == _NT - 1)
            def _():
                pltpu.sync_copy(
                    msg_sh.at[pl.ds((_NT - 1) * _RT0, _RTL)],
                    out_h.at[pl.ds(c * _N4 + (_NT - 1) * _RT0, _RTL)])

            if j + 1 < npc:
                plsc.subcore_barrier()

    return msg_kernel


_make_msg_kernel = functools.lru_cache(maxsize=None)(_make_msg_kernel)


def _stats(x3):
    G, _, K = x3.shape

    def body(x_ref, o_ref, acc):
        i = pl.program_id(1)

        @pl.when(i == 0)
        def _():
            acc[...] = jnp.zeros_like(acc)

        xb = x_ref[0]
        acc[0:1, :] += jnp.sum(xb, axis=0, keepdims=True)
        acc[1:2, :] += jnp.sum(xb * xb, axis=0, keepdims=True)

        @pl.when(i == _NRB - 1)
        def _():
            o_ref[0] = acc[...]

    return pl.pallas_call(
        body,
        grid=(G, _NRB),
        in_specs=[pl.BlockSpec((1, _ROWB, K), lambda g, i: (g, i, 0))],
        out_specs=pl.BlockSpec((1, 2, K), lambda g, i: (g, 0, 0)),
        out_shape=jax.ShapeDtypeStruct((G, 2, K), jnp.float32),
        scratch_shapes=[pltpu.VMEM((2, K), jnp.float32)],
    )(x3)


def _apply(blocks, ss_list, w_list, bias, dout):
    n_in = len(blocks)
    n_sub = len(w_list)

    def body(*refs):
        bl_refs = refs[:n_in]
        ss_refs = refs[n_in:n_in + n_sub]
        w_refs = refs[n_in + n_sub:n_in + 2 * n_sub]
        bias_ref = refs[n_in + 2 * n_sub]
        out_ref, st_ref, acc = refs[n_in + 2 * n_sub + 1:]
        i = pl.program_id(0)

        @pl.when(i == 0)
        def _():
            acc[...] = jnp.zeros_like(acc)

        subs = []
        for r, arr in zip(bl_refs, blocks):
            if arr.ndim == 3:
                for cc in range(arr.shape[0]):
                    subs.append(r[cc])
            else:
                subs.append(r[...])

        tot = None
        for xb, ssr, wr in zip(subs, ss_refs, w_refs):
            s = ssr[...]
            xn = xb * s[0:1, :] + s[1:2, :]
            e = jnp.where(xn > 0, xn, jnp.exp(xn) - 1.0)
            d = jnp.dot(e, wr[...], precision=lax.Precision.HIGHEST,
                        preferred_element_type=jnp.float32)
            tot = d if tot is None else tot + d
        tot = tot + bias_ref[...]
        out_ref[...] = tot
        acc[0:1, :] += jnp.sum(tot, axis=0, keepdims=True)
        acc[1:2, :] += jnp.sum(tot * tot, axis=0, keepdims=True)

        @pl.when(i == _NRB - 1)
        def _():
            st_ref[...] = acc[...]

    in_specs = []
    for arr in blocks:
        if arr.ndim == 3:
            Cc, _, Kc = arr.shape
            in_specs.append(
                pl.BlockSpec((Cc, _ROWB, Kc), lambda i: (0, i, 0)))
        else:
            K = arr.shape[1]
            in_specs.append(pl.BlockSpec((_ROWB, K), lambda i: (i, 0)))
    for s in ss_list:
        in_specs.append(pl.BlockSpec(s.shape, lambda i: (0, 0)))
    for w in w_list:
        in_specs.append(pl.BlockSpec(w.shape, lambda i: (0, 0)))
    in_specs.append(pl.BlockSpec((1, dout), lambda i: (0, 0)))

    out, st = pl.pallas_call(
        body,
        grid=(_NRB,),
        in_specs=in_specs,
        out_specs=[
            pl.BlockSpec((_ROWB, dout), lambda i: (i, 0)),
            pl.BlockSpec((2, dout), lambda i: (0, 0)),
        ],
        out_shape=[
            jax.ShapeDtypeStruct((_N, dout), jnp.float32),
            jax.ShapeDtypeStruct((2, dout), jnp.float32),
        ],
        scratch_shapes=[pltpu.VMEM((2, dout), jnp.float32)],
    )(*blocks, *ss_list, *w_list, bias)
    return out, st


def _ss(stats, gamma, beta):
    mean = stats[0] / _N
    var = stats[1] / _N - mean * mean
    scale = gamma * lax.rsqrt(var + 1e-5)
    return jnp.stack([scale, beta - mean * scale])


def _mol_conv_layer(x, xstats, edges, zeros, p, F):
    C = F // _CW
    KM = _NBT * _CW
    beg3, end3, bt3 = edges
    xcm = x.reshape(_N, C, _CW).transpose(1, 0, 2).reshape(C * _N, _CW)
    msg = _make_msg_kernel(C)(xcm, beg3, end3, bt3, zeros)
    msg3 = msg.reshape(C, _N, KM)
    mstats = _stats(msg3)

    W, gamma, beta = p["W"], p["gamma"], p["beta"]
    dout = W.shape[0]
    gm = gamma[F:].reshape(_NBT, C, _CW)
    bm = beta[F:].reshape(_NBT, C, _CW)
    Wm = W[:, F:].reshape(dout, _NBT, C, _CW)

    ss_list = [_ss(xstats, gamma[:F], beta[:F])]
    w_list = [W[:, :F].T]
    for c in range(C):
        ss_list.append(_ss(mstats[c], gm[:, c, :].reshape(KM),
                           bm[:, c, :].reshape(KM)))
        w_list.append(Wm[:, :, c, :].reshape(dout, KM).T)
    return _apply([x, msg3], ss_list, w_list, p["b"].reshape(1, dout), dout)


def _dense_apply(feats, p, dout):
    W, gamma, beta = p["W"], p["gamma"], p["beta"]
    blocks, ss_list, w_list = [], [], []
    off = 0
    for arr, st in feats:
        K = arr.shape[1]
        blocks.append(arr)
        ss_list.append(_ss(st, gamma[off:off + K], beta[off:off + K]))
        w_list.append(W[:, off:off + K].T)
        off += K
    return _apply(blocks, ss_list, w_list, p["b"].reshape(1, dout), dout)


def kernel(atom_features, bond_info, params):
    beg = bond_info[:, 0]
    end = bond_info[:, 1]
    bt = bond_info[:, 2]
    pad = _EP - _E
    beg3 = jnp.concatenate(
        [beg, jnp.zeros((pad,), jnp.int32)]).reshape(_NT, _NBATCH, _EB)
    end3 = jnp.concatenate(
        [end, jnp.full((pad,), _N, jnp.int32)]).reshape(_NT, _NBATCH, _EB)
    bt3 = jnp.concatenate(
        [bt, jnp.zeros((pad,), jnp.int32)]).reshape(_NT, _NBATCH, _EB)
    edges = (beg3, end3, bt3)
    zeros = jnp.zeros((_RTL, _CW), jnp.float32)

    x = atom_features
    xstats = _stats(x.reshape(1, _N, 128))[0]
    for p in params["causal"]:
        x, xstats = _mol_conv_layer(x, xstats, edges, zeros, p, 128)

    feats = [(x, xstats)]
    for lp in params["dense"]:
        bn, bnstats = _dense_apply(feats, lp["bottleneck"], 64)
        k, kstats = _mol_conv_layer(bn, bnstats, edges, zeros, lp["conv"], 64)
        feats.append((k, kstats))

    out, _ = _dense_apply(feats, params["output"], 128)
    return out


# speedup vs baseline: 6.4135x; 1.4145x over previous
import functools

import jax
import jax.numpy as jnp
from jax import lax
from jax.experimental import pallas as pl
from jax.experimental.pallas import tpu as pltpu
from jax.experimental.pallas import tpu_sc as plsc

_N = 10000
_E = 320000
_NBT = 4
_N4 = _NBT * _N
_N4P = _N4 + 16
_NT = 16
_EB = 128
_ET = 20224
_NBATCH = _ET // _EB
_NBH = _NBATCH // 2
_EP = _NT * _ET
_RT0 = 2496
_RTL = _N4 - (_NT - 1) * _RT0
_CW = 32
_ROWB = 1000
_NRB = _N // _ROWB


_XR0 = 624
_XRL = _N - (_NT - 1) * _XR0


def _make_msg_kernel(C):
    npc = C // 2
    mesh = plsc.VectorSubcoreMesh(core_axis_name="c", subcore_axis_name="s",
                                  num_cores=2, num_subcores=_NT)

    @functools.partial(
        pl.kernel,
        out_type=jax.ShapeDtypeStruct((C * _N4, _CW), jnp.float32),
        mesh=mesh,
        compiler_params=pltpu.CompilerParams(use_tc_tiling_on_sc=False),
        scratch_types=[
            pltpu.VMEM_SHARED((_N4P, _CW), jnp.float32),
            pltpu.VMEM_SHARED((_N, _CW), jnp.float32),
            pltpu.VMEM((_NBH, _EB), jnp.int32),
            pltpu.VMEM((_NBH, _EB), jnp.int32),
            pltpu.VMEM((2, _EB, _CW), jnp.float32),
            pltpu.SemaphoreType.DMA,
            pltpu.SemaphoreType.DMA,
        ],
    )
    def msg_kernel(xcm, beg_h, end_h, bt_h, zeros_h, out_h,
                   msg_sh, x_sp, beg_v, dst_v, rows_v, sem0, sem1):
        cid = lax.axis_index("c")
        sid = lax.axis_index("s")

        for j in range(npc):
            c = cid * npc + j

            @pl.when(sid < _NT - 1)
            def _():
                pltpu.sync_copy(
                    xcm.at[pl.ds(c * _N + sid * _XR0, _XR0)],
                    x_sp.at[pl.ds(sid * _XR0, _XR0)])

            @pl.when(sid == _NT - 1)
            def _():
                pltpu.sync_copy(
                    xcm.at[pl.ds(c * _N + (_NT - 1) * _XR0, _XRL)],
                    x_sp.at[pl.ds((_NT - 1) * _XR0, _XRL)])

            @pl.when(sid < _NT - 1)
            def _():
                pltpu.sync_copy(zeros_h.at[pl.ds(0, _RT0)],
                                msg_sh.at[pl.ds(sid * _RT0, _RT0)])

            @pl.when(sid == _NT - 1)
            def _():
                pltpu.sync_copy(zeros_h,
                                msg_sh.at[pl.ds((_NT - 1) * _RT0, _RTL)])

            plsc.subcore_barrier()

            for hb in (0, _NBH):
                pltpu.sync_copy(end_h.at[sid].at[pl.ds(hb, _NBH)], dst_v)
                pltpu.sync_copy(bt_h.at[sid].at[pl.ds(hb, _NBH)], beg_v)

                @pl.loop(0, _NBH)
                def _(b):
                    for i in range(_EB // 16):
                        s = pl.ds(i * 16, 16)
                        dst_v[b, s] = dst_v[b, s] * _NBT + beg_v[b, s]

                pltpu.sync_copy(beg_h.at[sid].at[pl.ds(hb, _NBH)], beg_v)

                pltpu.async_copy(x_sp.at[beg_v.at[0]], rows_v.at[0], sem0)

                @pl.loop(0, _NBH // 2)
                def _(k):
                    b0 = k * 2
                    pltpu.make_async_copy(
                        x_sp.at[beg_v.at[b0]], rows_v.at[0], sem0).wait()
                    pltpu.async_copy(
                        x_sp.at[beg_v.at[b0 + 1]], rows_v.at[1], sem1)
                    pltpu.sync_copy(
                        rows_v.at[0], msg_sh.at[dst_v.at[b0]], add=True)
                    pltpu.make_async_copy(
                        x_sp.at[beg_v.at[b0 + 1]], rows_v.at[1], sem1).wait()
                    pltpu.async_copy(
                        x_sp.at[beg_v.at[b0 + 2]], rows_v.at[0], sem0)
                    pltpu.sync_copy(
                        rows_v.at[1], msg_sh.at[dst_v.at[b0 + 1]], add=True)

                bl = _NBH - 1
                pltpu.make_async_copy(
                    x_sp.at[beg_v.at[bl]], rows_v.at[0], sem0).wait()
                pltpu.sync_copy(rows_v.at[0], msg_sh.at[dst_v.at[bl]],
                                add=True)

            plsc.subcore_barrier()

            @pl.when(sid < _NT - 1)
            def _():
                pltpu.sync_copy(
                    msg_sh.at[pl.ds(sid * _RT0, _RT0)],
                    out_h.at[pl.ds(c * _N4 + sid * _RT0, _RT0)])

            @pl.when(sid == _NT - 1)
            def _():
                pltpu.sync_copy(
                    msg_sh.at[pl.ds((_NT - 1) * _RT0, _RTL)],
                    out_h.at[pl.ds(c * _N4 + (_NT - 1) * _RT0, _RTL)])

            if j + 1 < npc:
                plsc.subcore_barrier()

    return msg_kernel


_make_msg_kernel = functools.lru_cache(maxsize=None)(_make_msg_kernel)


def _stats(x3):
    G, _, K = x3.shape

    def body(x_ref, o_ref, acc):
        i = pl.program_id(1)

        @pl.when(i == 0)
        def _():
            acc[...] = jnp.zeros_like(acc)

        xb = x_ref[0]
        acc[0:1, :] += jnp.sum(xb, axis=0, keepdims=True)
        acc[1:2, :] += jnp.sum(xb * xb, axis=0, keepdims=True)

        @pl.when(i == _NRB - 1)
        def _():
            o_ref[0] = acc[...]

    return pl.pallas_call(
        body,
        grid=(G, _NRB),
        in_specs=[pl.BlockSpec((1, _ROWB, K), lambda g, i: (g, i, 0))],
        out_specs=pl.BlockSpec((1, 2, K), lambda g, i: (g, 0, 0)),
        out_shape=jax.ShapeDtypeStruct((G, 2, K), jnp.float32),
        scratch_shapes=[pltpu.VMEM((2, K), jnp.float32)],
    )(x3)


def _apply(blocks, ss_list, w_list, bias, dout):
    n_in = len(blocks)
    n_sub = len(w_list)

    def body(*refs):
        bl_refs = refs[:n_in]
        ss_refs = refs[n_in:n_in + n_sub]
        w_refs = refs[n_in + n_sub:n_in + 2 * n_sub]
        bias_ref = refs[n_in + 2 * n_sub]
        out_ref, st_ref, acc = refs[n_in + 2 * n_sub + 1:]
        i = pl.program_id(0)

        @pl.when(i == 0)
        def _():
            acc[...] = jnp.zeros_like(acc)

        subs = []
        for r, arr in zip(bl_refs, blocks):
            if arr.ndim == 3:
                for cc in range(arr.shape[0]):
                    subs.append(r[cc])
            else:
                subs.append(r[...])

        tot = None
        for xb, ssr, wr in zip(subs, ss_refs, w_refs):
            s = ssr[...]
            xn = xb * s[0:1, :] + s[1:2, :]
            e = jnp.where(xn > 0, xn, jnp.exp(xn) - 1.0)
            d = jnp.dot(e, wr[...], precision=lax.Precision.HIGHEST,
                        preferred_element_type=jnp.float32)
            tot = d if tot is None else tot + d
        tot = tot + bias_ref[...]
        out_ref[...] = tot
        acc[0:1, :] += jnp.sum(tot, axis=0, keepdims=True)
        acc[1:2, :] += jnp.sum(tot * tot, axis=0, keepdims=True)

        @pl.when(i == _NRB - 1)
        def _():
            st_ref[...] = acc[...]

    in_specs = []
    for arr in blocks:
        if arr.ndim == 3:
            Cc, _, Kc = arr.shape
            in_specs.append(
                pl.BlockSpec((Cc, _ROWB, Kc), lambda i: (0, i, 0)))
        else:
            K = arr.shape[1]
            in_specs.append(pl.BlockSpec((_ROWB, K), lambda i: (i, 0)))
    for s in ss_list:
        in_specs.append(pl.BlockSpec(s.shape, lambda i: (0, 0)))
    for w in w_list:
        in_specs.append(pl.BlockSpec(w.shape, lambda i: (0, 0)))
    in_specs.append(pl.BlockSpec((1, dout), lambda i: (0, 0)))

    out, st = pl.pallas_call(
        body,
        grid=(_NRB,),
        in_specs=in_specs,
        out_specs=[
            pl.BlockSpec((_ROWB, dout), lambda i: (i, 0)),
            pl.BlockSpec((2, dout), lambda i: (0, 0)),
        ],
        out_shape=[
            jax.ShapeDtypeStruct((_N, dout), jnp.float32),
            jax.ShapeDtypeStruct((2, dout), jnp.float32),
        ],
        scratch_shapes=[pltpu.VMEM((2, dout), jnp.float32)],
    )(*blocks, *ss_list, *w_list, bias)
    return out, st


def _ss(stats, gamma, beta):
    mean = stats[0] / _N
    var = stats[1] / _N - mean * mean
    scale = gamma * lax.rsqrt(var + 1e-5)
    return jnp.stack([scale, beta - mean * scale])


def _mol_conv_layer(x, xstats, edges, zeros, p, F):
    C = F // _CW
    KM = _NBT * _CW
    beg3, end3, bt3 = edges
    xcm = x.reshape(_N, C, _CW).transpose(1, 0, 2).reshape(C * _N, _CW)
    msg = _make_msg_kernel(C)(xcm, beg3, end3, bt3, zeros)
    msg3 = msg.reshape(C, _N, KM)
    mstats = _stats(msg3)

    W, gamma, beta = p["W"], p["gamma"], p["beta"]
    dout = W.shape[0]
    gm = gamma[F:].reshape(_NBT, C, _CW)
    bm = beta[F:].reshape(_NBT, C, _CW)
    Wm = W[:, F:].reshape(dout, _NBT, C, _CW)

    ss_list = [_ss(xstats, gamma[:F], beta[:F])]
    w_list = [W[:, :F].T]
    for c in range(C):
        ss_list.append(_ss(mstats[c], gm[:, c, :].reshape(KM),
                           bm[:, c, :].reshape(KM)))
        w_list.append(Wm[:, :, c, :].reshape(dout, KM).T)
    return _apply([x, msg3], ss_list, w_list, p["b"].reshape(1, dout), dout)


def _dense_apply(feats, p, dout):
    W, gamma, beta = p["W"], p["gamma"], p["beta"]
    blocks, ss_list, w_list = [], [], []
    off = 0
    for arr, st in feats:
        K = arr.shape[1]
        blocks.append(arr)
        ss_list.append(_ss(st, gamma[off:off + K], beta[off:off + K]))
        w_list.append(W[:, off:off + K].T)
        off += K
    return _apply(blocks, ss_list, w_list, p["b"].reshape(1, dout), dout)


def kernel(atom_features, bond_info, params):
    beg = bond_info[:, 0]
    end = bond_info[:, 1]
    bt = bond_info[:, 2]
    pad = _EP - _E
    beg3 = jnp.concatenate(
        [beg, jnp.zeros((pad,), jnp.int32)]).reshape(_NT, _NBATCH, _EB)
    end3 = jnp.concatenate(
        [end, jnp.full((pad,), _N, jnp.int32)]).reshape(_NT, _NBATCH, _EB)
    bt3 = jnp.concatenate(
        [bt, jnp.zeros((pad,), jnp.int32)]).reshape(_NT, _NBATCH, _EB)
    edges = (beg3, end3, bt3)
    zeros = jnp.zeros((_RTL, _CW), jnp.float32)

    x = atom_features
    xstats = _stats(x.reshape(1, _N, 128))[0]
    for p in params["causal"]:
        x, xstats = _mol_conv_layer(x, xstats, edges, zeros, p, 128)

    feats = [(x, xstats)]
    for lp in params["dense"]:
        bn, bnstats = _dense_apply(feats, lp["bottleneck"], 64)
        k, kstats = _mol_conv_layer(bn, bnstats, edges, zeros, lp["conv"], 64)
        feats.append((k, kstats))

    out, _ = _dense_apply(feats, params["output"], 128)
    return out


# speedup vs baseline: 6.7662x; 1.0550x over previous
import functools

import jax
import jax.numpy as jnp
from jax import lax
from jax.experimental import pallas as pl
from jax.experimental.pallas import tpu as pltpu
from jax.experimental.pallas import tpu_sc as plsc

_N = 10000
_E = 320000
_NBT = 4
_N4 = _NBT * _N
_N4P = _N4 + 16
_NT = 16
_EB = 128
_ET = 20224
_NBATCH = _ET // _EB
_NBH = _NBATCH // 2
_EP = _NT * _ET
_RT0 = 2496
_RTL = _N4 - (_NT - 1) * _RT0
_CW = 32
_ROWB = 1000
_NRB = _N // _ROWB


_XR0 = 624
_XRL = _N - (_NT - 1) * _XR0


def _make_msg_kernel(C):
    npc = C // 2
    mesh = plsc.VectorSubcoreMesh(core_axis_name="c", subcore_axis_name="s",
                                  num_cores=2, num_subcores=_NT)

    @functools.partial(
        pl.kernel,
        out_type=jax.ShapeDtypeStruct((C * _N4, _CW), jnp.float32),
        mesh=mesh,
        compiler_params=pltpu.CompilerParams(use_tc_tiling_on_sc=False),
        scratch_types=[
            pltpu.VMEM_SHARED((_N4P, _CW), jnp.float32),
            pltpu.VMEM_SHARED((_N, _CW), jnp.float32),
            pltpu.VMEM((_NBH, _EB), jnp.int32),
            pltpu.VMEM((_NBH, _EB), jnp.int32),
            pltpu.VMEM((2, _EB, _CW), jnp.float32),
            pltpu.SemaphoreType.DMA,
            pltpu.SemaphoreType.DMA,
        ],
    )
    def msg_kernel(xcm, beg_h, end_h, bt_h, zeros_h, out_h,
                   msg_sh, x_sp, beg_v, dst_v, rows_v, sem0, sem1):
        cid = lax.axis_index("c")
        sid = lax.axis_index("s")

        for j in range(npc):
            c = cid * npc + j

            @pl.when(sid < _NT - 1)
            def _():
                pltpu.sync_copy(
                    xcm.at[pl.ds(c * _N + sid * _XR0, _XR0)],
                    x_sp.at[pl.ds(sid * _XR0, _XR0)])

            @pl.when(sid == _NT - 1)
            def _():
                pltpu.sync_copy(
                    xcm.at[pl.ds(c * _N + (_NT - 1) * _XR0, _XRL)],
                    x_sp.at[pl.ds((_NT - 1) * _XR0, _XRL)])

            @pl.when(sid < _NT - 1)
            def _():
                pltpu.sync_copy(zeros_h.at[pl.ds(0, _RT0)],
                                msg_sh.at[pl.ds(sid * _RT0, _RT0)])

            @pl.when(sid == _NT - 1)
            def _():
                pltpu.sync_copy(zeros_h,
                                msg_sh.at[pl.ds((_NT - 1) * _RT0, _RTL)])

            plsc.subcore_barrier()

            for hb in (0, _NBH):
                pltpu.sync_copy(end_h.at[sid].at[pl.ds(hb, _NBH)], dst_v)
                pltpu.sync_copy(bt_h.at[sid].at[pl.ds(hb, _NBH)], beg_v)

                @pl.loop(0, _NBH)
                def _(b):
                    for i in range(_EB // 16):
                        s = pl.ds(i * 16, 16)
                        dst_v[b, s] = dst_v[b, s] * _NBT + beg_v[b, s]

                pltpu.sync_copy(beg_h.at[sid].at[pl.ds(hb, _NBH)], beg_v)

                pltpu.async_copy(x_sp.at[beg_v.at[0]], rows_v.at[0], sem0)

                @pl.loop(0, _NBH // 2)
                def _(k):
                    b0 = k * 2
                    pltpu.make_async_copy(
                        x_sp.at[beg_v.at[b0]], rows_v.at[0], sem0).wait()
                    pltpu.async_copy(
                        x_sp.at[beg_v.at[b0 + 1]], rows_v.at[1], sem1)
                    pltpu.sync_copy(
                        rows_v.at[0], msg_sh.at[dst_v.at[b0]], add=True)
                    pltpu.make_async_copy(
                        x_sp.at[beg_v.at[b0 + 1]], rows_v.at[1], sem1).wait()
                    pltpu.async_copy(
                        x_sp.at[beg_v.at[b0 + 2]], rows_v.at[0], sem0)
                    pltpu.sync_copy(
                        rows_v.at[1], msg_sh.at[dst_v.at[b0 + 1]], add=True)

                bl = _NBH - 1
                pltpu.make_async_copy(
                    x_sp.at[beg_v.at[bl]], rows_v.at[0], sem0).wait()
                pltpu.sync_copy(rows_v.at[0], msg_sh.at[dst_v.at[bl]],
                                add=True)

            plsc.subcore_barrier()

            @pl.when(sid < _NT - 1)
            def _():
                pltpu.sync_copy(
                    msg_sh.at[pl.ds(sid * _RT0, _RT0)],
                    out_h.at[pl.ds(c * _N4 + sid * _RT0, _RT0)])

            @pl.when(sid == _NT - 1)
            def _():
                pltpu.sync_copy(
                    msg_sh.at[pl.ds((_NT - 1) * _RT0, _RTL)],
                    out_h.at[pl.ds(c * _N4 + (_NT - 1) * _RT0, _RTL)])

            if j + 1 < npc:
                plsc.subcore_barrier()

    return msg_kernel


_make_msg_kernel = functools.lru_cache(maxsize=None)(_make_msg_kernel)


def _stats(x3):
    G, _, K = x3.shape

    def body(x_ref, o_ref, acc):
        i = pl.program_id(1)

        @pl.when(i == 0)
        def _():
            acc[...] = jnp.zeros_like(acc)

        xb = x_ref[0]
        acc[0:1, :] += jnp.sum(xb, axis=0, keepdims=True)
        acc[1:2, :] += jnp.sum(xb * xb, axis=0, keepdims=True)

        @pl.when(i == _NRB - 1)
        def _():
            o_ref[0] = acc[...]

    return pl.pallas_call(
        body,
        grid=(G, _NRB),
        in_specs=[pl.BlockSpec((1, _ROWB, K), lambda g, i: (g, i, 0))],
        out_specs=pl.BlockSpec((1, 2, K), lambda g, i: (g, 0, 0)),
        out_shape=jax.ShapeDtypeStruct((G, 2, K), jnp.float32),
        scratch_shapes=[pltpu.VMEM((2, K), jnp.float32)],
    )(x3)


def _apply(blocks, ss_list, w_list, bias, dout, out_width=None,
           alias_out=None, alias_off=0):
    arrs = []
    widths = []
    for b in blocks:
        if isinstance(b, tuple):
            arrs.append(b[0]); widths.append(b[1])
        else:
            arrs.append(b); widths.append(None)
    n_in = len(arrs)
    n_sub = len(w_list)

    def body(*refs):
        bl_refs = refs[:n_in]
        ss_refs = refs[n_in:n_in + n_sub]
        w_refs = refs[n_in + n_sub:n_in + 2 * n_sub]
        bias_ref = refs[n_in + 2 * n_sub]
        k = n_in + 2 * n_sub + 1
        alias_ref = None
        if alias_out is not None:
            alias_ref = refs[k]
            k += 1
        out_ref, st_ref, acc = refs[k:]
        i = pl.program_id(0)

        @pl.when(i == 0)
        def _():
            acc[...] = jnp.zeros_like(acc)

        subs = []
        for r, arr, kw in zip(bl_refs, arrs, widths):
            if arr.ndim == 3:
                for cc in range(arr.shape[0]):
                    subs.append(r[cc])
            elif kw is not None and kw < arr.shape[1]:
                subs.append(r[...][:, :kw])
            else:
                subs.append(r[...])

        tot = None
        for xb, ssr, wr in zip(subs, ss_refs, w_refs):
            s = ssr[...]
            xn = xb * s[0:1, :] + s[1:2, :]
            e = jnp.where(xn > 0, xn, jnp.exp(xn) - 1.0)
            d = jnp.dot(e, wr[...], precision=lax.Precision.HIGHEST,
                        preferred_element_type=jnp.float32)
            tot = d if tot is None else tot + d
        tot = tot + bias_ref[...]
        if alias_out is None:
            out_ref[...] = tot
        else:
            ob = alias_ref[...]
            parts = []
            if alias_off > 0:
                parts.append(ob[:, :alias_off])
            parts.append(tot)
            if alias_off + dout < ob.shape[1]:
                parts.append(ob[:, alias_off + dout:])
            out_ref[...] = (parts[0] if len(parts) == 1
                            else jnp.concatenate(parts, axis=1))
        acc[0:1, :] += jnp.sum(tot, axis=0, keepdims=True)
        acc[1:2, :] += jnp.sum(tot * tot, axis=0, keepdims=True)

        @pl.when(i == _NRB - 1)
        def _():
            st_ref[...] = acc[...]

    in_specs = []
    for arr, kw in zip(arrs, widths):
        if arr.ndim == 3:
            Cc, _, Kc = arr.shape
            in_specs.append(
                pl.BlockSpec((Cc, _ROWB, Kc), lambda i: (0, i, 0)))
        else:
            in_specs.append(
                pl.BlockSpec((_ROWB, arr.shape[1]), lambda i: (i, 0)))
    for s in ss_list:
        in_specs.append(pl.BlockSpec(s.shape, lambda i: (0, 0)))
    for w in w_list:
        in_specs.append(pl.BlockSpec(w.shape, lambda i: (0, 0)))
    in_specs.append(pl.BlockSpec((1, dout), lambda i: (0, 0)))

    args = list(arrs) + list(ss_list) + list(w_list) + [bias]
    io_aliases = {}
    if alias_out is not None:
        OW = alias_out.shape[1]
        in_specs.append(pl.BlockSpec((_ROWB, 128), lambda i: (i, 1)))
        args.append(alias_out)
        io_aliases = {len(args) - 1: 0}
        out0_shape = jax.ShapeDtypeStruct((_N, OW), jnp.float32)
        out0_spec = pl.BlockSpec((_ROWB, 128), lambda i: (i, 1))
    else:
        OW = out_width if out_width is not None else dout
        out0_shape = jax.ShapeDtypeStruct((_N, OW), jnp.float32)
        out0_spec = pl.BlockSpec((_ROWB, dout), lambda i: (i, 0))

    out, st = pl.pallas_call(
        body,
        grid=(_NRB,),
        in_specs=in_specs,
        out_specs=[
            out0_spec,
            pl.BlockSpec((2, dout), lambda i: (0, 0)),
        ],
        out_shape=[
            out0_shape,
            jax.ShapeDtypeStruct((2, dout), jnp.float32),
        ],
        scratch_shapes=[pltpu.VMEM((2, dout), jnp.float32)],
        input_output_aliases=io_aliases,
    )(*args)
    return out, st


def _ss(stats, gamma, beta):
    mean = stats[0] / _N
    var = stats[1] / _N - mean * mean
    scale = gamma * lax.rsqrt(var + 1e-5)
    return jnp.stack([scale, beta - mean * scale])


def _mol_conv_layer(x, xstats, edges, zeros, p, F, **kw):
    C = F // _CW
    KM = _NBT * _CW
    beg3, end3, bt3 = edges
    xcm = x.reshape(_N, C, _CW).transpose(1, 0, 2).reshape(C * _N, _CW)
    msg = _make_msg_kernel(C)(xcm, beg3, end3, bt3, zeros)
    msg3 = msg.reshape(C, _N, KM)
    mstats = _stats(msg3)

    W, gamma, beta = p["W"], p["gamma"], p["beta"]
    dout = W.shape[0]
    gm = gamma[F:].reshape(_NBT, C, _CW)
    bm = beta[F:].reshape(_NBT, C, _CW)
    Wm = W[:, F:].reshape(dout, _NBT, C, _CW)

    ss_list = [_ss(xstats, gamma[:F], beta[:F])]
    w_list = [W[:, :F].T]
    for c in range(C):
        ss_list.append(_ss(mstats[c], gm[:, c, :].reshape(KM),
                           bm[:, c, :].reshape(KM)))
        w_list.append(Wm[:, :, c, :].reshape(dout, KM).T)
    return _apply([x, msg3], ss_list, w_list, p["b"].reshape(1, dout), dout,
                  **kw)


def _dense_apply(cat, K, stats, p, dout, **kw):
    W, gamma, beta = p["W"], p["gamma"], p["beta"]
    ss = _ss(stats, gamma, beta)
    return _apply([(cat, K)], [ss], [W.T], p["b"].reshape(1, dout), dout, **kw)


def kernel(atom_features, bond_info, params):
    beg = bond_info[:, 0]
    end = bond_info[:, 1]
    bt = bond_info[:, 2]
    pad = _EP - _E
    beg3 = jnp.concatenate(
        [beg, jnp.zeros((pad,), jnp.int32)]).reshape(_NT, _NBATCH, _EB)
    end3 = jnp.concatenate(
        [end, jnp.full((pad,), _N, jnp.int32)]).reshape(_NT, _NBATCH, _EB)
    bt3 = jnp.concatenate(
        [bt, jnp.zeros((pad,), jnp.int32)]).reshape(_NT, _NBATCH, _EB)
    edges = (beg3, end3, bt3)
    zeros = jnp.zeros((_RTL, _CW), jnp.float32)

    x = atom_features
    xstats = _stats(x.reshape(1, _N, 128))[0]
    p0, p1 = params["causal"]
    x, xstats = _mol_conv_layer(x, xstats, edges, zeros, p0, 128)
    cat, hstats = _mol_conv_layer(x, xstats, edges, zeros, p1, 128,
                                  out_width=128 + 4 * 32)

    stats_list = [hstats]
    for li, lp in enumerate(params["dense"]):
        K = 128 + 32 * li
        cstats = jnp.concatenate(stats_list, axis=1)
        bn, bnstats = _dense_apply(cat, K, cstats, lp["bottleneck"], 64)
        cat, kstats = _mol_conv_layer(bn, bnstats, edges, zeros,
                                      lp["conv"], 64,
                                      alias_out=cat, alias_off=32 * li)
        stats_list.append(kstats)

    cstats = jnp.concatenate(stats_list, axis=1)
    out, _ = _dense_apply(cat, 256, cstats, params["output"], 128)
    return out


# speedup vs baseline: 7.5738x; 1.1194x over previous
import functools

import jax
import jax.numpy as jnp
from jax import lax
from jax.experimental import pallas as pl
from jax.experimental.pallas import tpu as pltpu
from jax.experimental.pallas import tpu_sc as plsc

_N = 10000
_E = 320000
_NBT = 4
_N4 = _NBT * _N
_N4P = _N4 + 16
_NT = 16
_EB = 128
_ET = 20224
_NBATCH = _ET // _EB
_NBH = _NBATCH // 2
_EP = _NT * _ET
_RT0 = 2496
_RTL = _N4 - (_NT - 1) * _RT0
_CW = 32
_ROWB = 1000
_NRB = _N // _ROWB


_XR0 = 624
_XRL = _N - (_NT - 1) * _XR0


def _make_msg_kernel(C):
    npc = C // 2
    mesh = plsc.VectorSubcoreMesh(core_axis_name="c", subcore_axis_name="s",
                                  num_cores=2, num_subcores=_NT)

    @functools.partial(
        pl.kernel,
        out_type=jax.ShapeDtypeStruct((C * _N4, _CW), jnp.float32),
        mesh=mesh,
        compiler_params=pltpu.CompilerParams(use_tc_tiling_on_sc=False),
        scratch_types=[
            pltpu.VMEM_SHARED((_N4P, _CW), jnp.float32),
            pltpu.VMEM_SHARED((_N, _CW), jnp.float32),
            pltpu.VMEM((_NBH, _EB), jnp.int32),
            pltpu.VMEM((_NBH, _EB), jnp.int32),
            pltpu.VMEM((2, _EB, _CW), jnp.float32),
            pltpu.SemaphoreType.DMA,
            pltpu.SemaphoreType.DMA,
        ],
    )
    def msg_kernel(xcm, beg_h, end_h, bt_h, zeros_h, out_h,
                   msg_sh, x_sp, beg_v, dst_v, rows_v, sem0, sem1):
        cid = lax.axis_index("c")
        sid = lax.axis_index("s")

        for j in range(npc):
            c = cid * npc + j

            @pl.when(sid < _NT - 1)
            def _():
                pltpu.sync_copy(
                    xcm.at[pl.ds(c * _N + sid * _XR0, _XR0)],
                    x_sp.at[pl.ds(sid * _XR0, _XR0)])

            @pl.when(sid == _NT - 1)
            def _():
                pltpu.sync_copy(
                    xcm.at[pl.ds(c * _N + (_NT - 1) * _XR0, _XRL)],
                    x_sp.at[pl.ds((_NT - 1) * _XR0, _XRL)])

            @pl.when(sid < _NT - 1)
            def _():
                pltpu.sync_copy(zeros_h.at[pl.ds(0, _RT0)],
                                msg_sh.at[pl.ds(sid * _RT0, _RT0)])

            @pl.when(sid == _NT - 1)
            def _():
                pltpu.sync_copy(zeros_h,
                                msg_sh.at[pl.ds((_NT - 1) * _RT0, _RTL)])

            plsc.subcore_barrier()

            for hb in (0, _NBH):
                pltpu.sync_copy(end_h.at[sid].at[pl.ds(hb, _NBH)], dst_v)
                pltpu.sync_copy(bt_h.at[sid].at[pl.ds(hb, _NBH)], beg_v)

                @pl.loop(0, _NBH)
                def _(b):
                    for i in range(_EB // 16):
                        s = pl.ds(i * 16, 16)
                        dst_v[b, s] = dst_v[b, s] * _NBT + beg_v[b, s]

                pltpu.sync_copy(beg_h.at[sid].at[pl.ds(hb, _NBH)], beg_v)

                pltpu.async_copy(x_sp.at[beg_v.at[0]], rows_v.at[0], sem0)

                @pl.loop(0, _NBH // 2)
                def _(k):
                    b0 = k * 2
                    pltpu.make_async_copy(
                        x_sp.at[beg_v.at[b0]], rows_v.at[0], sem0).wait()
                    pltpu.async_copy(
                        x_sp.at[beg_v.at[b0 + 1]], rows_v.at[1], sem1)
                    pltpu.sync_copy(
                        rows_v.at[0], msg_sh.at[dst_v.at[b0]], add=True)
                    pltpu.make_async_copy(
                        x_sp.at[beg_v.at[b0 + 1]], rows_v.at[1], sem1).wait()
                    pltpu.async_copy(
                        x_sp.at[beg_v.at[b0 + 2]], rows_v.at[0], sem0)
                    pltpu.sync_copy(
                        rows_v.at[1], msg_sh.at[dst_v.at[b0 + 1]], add=True)

                bl = _NBH - 1
                pltpu.make_async_copy(
                    x_sp.at[beg_v.at[bl]], rows_v.at[0], sem0).wait()
                pltpu.sync_copy(rows_v.at[0], msg_sh.at[dst_v.at[bl]],
                                add=True)

            plsc.subcore_barrier()

            @pl.when(sid < _NT - 1)
            def _():
                pltpu.sync_copy(
                    msg_sh.at[pl.ds(sid * _RT0, _RT0)],
                    out_h.at[pl.ds(c * _N4 + sid * _RT0, _RT0)])

            @pl.when(sid == _NT - 1)
            def _():
                pltpu.sync_copy(
                    msg_sh.at[pl.ds((_NT - 1) * _RT0, _RTL)],
                    out_h.at[pl.ds(c * _N4 + (_NT - 1) * _RT0, _RTL)])

            if j + 1 < npc:
                plsc.subcore_barrier()

    return msg_kernel


_make_msg_kernel = functools.lru_cache(maxsize=None)(_make_msg_kernel)


def _stats(x3):
    G, _, K = x3.shape

    def body(x_ref, o_ref, acc):
        i = pl.program_id(1)

        @pl.when(i == 0)
        def _():
            acc[...] = jnp.zeros_like(acc)

        xb = x_ref[0]
        acc[0:1, :] += jnp.sum(xb, axis=0, keepdims=True)
        acc[1:2, :] += jnp.sum(xb * xb, axis=0, keepdims=True)

        @pl.when(i == _NRB - 1)
        def _():
            o_ref[0] = acc[...]

    return pl.pallas_call(
        body,
        grid=(G, _NRB),
        in_specs=[pl.BlockSpec((1, _ROWB, K), lambda g, i: (g, i, 0))],
        out_specs=pl.BlockSpec((1, 2, K), lambda g, i: (g, 0, 0)),
        out_shape=jax.ShapeDtypeStruct((G, 2, K), jnp.float32),
        scratch_shapes=[pltpu.VMEM((2, K), jnp.float32)],
    )(x3)


def _apply(blocks, ss_list, w_list, bias, dout, out_width=None,
           alias_out=None, alias_off=0):
    arrs = []
    widths = []
    for b in blocks:
        if isinstance(b, tuple):
            arrs.append(b[0]); widths.append(b[1])
        else:
            arrs.append(b); widths.append(None)
    n_in = len(arrs)
    n_sub = len(w_list)

    def body(*refs):
        bl_refs = refs[:n_in]
        ss_refs = refs[n_in:n_in + n_sub]
        w_refs = refs[n_in + n_sub:n_in + 2 * n_sub]
        bias_ref = refs[n_in + 2 * n_sub]
        k = n_in + 2 * n_sub + 1
        alias_ref = None
        if alias_out is not None:
            alias_ref = refs[k]
            k += 1
        out_ref, st_ref, acc = refs[k:]
        i = pl.program_id(0)

        @pl.when(i == 0)
        def _():
            acc[...] = jnp.zeros_like(acc)

        subs = []
        for r, arr, kw in zip(bl_refs, arrs, widths):
            if arr.ndim == 3:
                for cc in range(arr.shape[0]):
                    subs.append(r[cc])
            elif kw is not None and kw < arr.shape[1]:
                subs.append(r[...][:, :kw])
            else:
                subs.append(r[...])

        tot = None
        for xb, ssr, wr in zip(subs, ss_refs, w_refs):
            s = ssr[...]
            xn = xb * s[0:1, :] + s[1:2, :]
            e = jnp.where(xn > 0, xn, jnp.exp(xn) - 1.0)
            d = jnp.dot(e, wr[...], preferred_element_type=jnp.float32)
            tot = d if tot is None else tot + d
        tot = tot + bias_ref[...]
        if alias_out is None:
            out_ref[...] = tot
        else:
            ob = alias_ref[...]
            parts = []
            if alias_off > 0:
                parts.append(ob[:, :alias_off])
            parts.append(tot)
            if alias_off + dout < ob.shape[1]:
                parts.append(ob[:, alias_off + dout:])
            out_ref[...] = (parts[0] if len(parts) == 1
                            else jnp.concatenate(parts, axis=1))
        acc[0:1, :] += jnp.sum(tot, axis=0, keepdims=True)
        acc[1:2, :] += jnp.sum(tot * tot, axis=0, keepdims=True)

        @pl.when(i == _NRB - 1)
        def _():
            st_ref[...] = acc[...]

    in_specs = []
    for arr, kw in zip(arrs, widths):
        if arr.ndim == 3:
            Cc, _, Kc = arr.shape
            in_specs.append(
                pl.BlockSpec((Cc, _ROWB, Kc), lambda i: (0, i, 0)))
        else:
            in_specs.append(
                pl.BlockSpec((_ROWB, arr.shape[1]), lambda i: (i, 0)))
    for s in ss_list:
        in_specs.append(pl.BlockSpec(s.shape, lambda i: (0, 0)))
    for w in w_list:
        in_specs.append(pl.BlockSpec(w.shape, lambda i: (0, 0)))
    in_specs.append(pl.BlockSpec((1, dout), lambda i: (0, 0)))

    args = list(arrs) + list(ss_list) + list(w_list) + [bias]
    io_aliases = {}
    if alias_out is not None:
        OW = alias_out.shape[1]
        in_specs.append(pl.BlockSpec((_ROWB, 128), lambda i: (i, 1)))
        args.append(alias_out)
        io_aliases = {len(args) - 1: 0}
        out0_shape = jax.ShapeDtypeStruct((_N, OW), jnp.float32)
        out0_spec = pl.BlockSpec((_ROWB, 128), lambda i: (i, 1))
    else:
        OW = out_width if out_width is not None else dout
        out0_shape = jax.ShapeDtypeStruct((_N, OW), jnp.float32)
        out0_spec = pl.BlockSpec((_ROWB, dout), lambda i: (i, 0))

    out, st = pl.pallas_call(
        body,
        grid=(_NRB,),
        in_specs=in_specs,
        out_specs=[
            out0_spec,
            pl.BlockSpec((2, dout), lambda i: (0, 0)),
        ],
        out_shape=[
            out0_shape,
            jax.ShapeDtypeStruct((2, dout), jnp.float32),
        ],
        scratch_shapes=[pltpu.VMEM((2, dout), jnp.float32)],
        input_output_aliases=io_aliases,
    )(*args)
    return out, st


def _ss(stats, gamma, beta):
    mean = stats[0] / _N
    var = stats[1] / _N - mean * mean
    scale = gamma * lax.rsqrt(var + 1e-5)
    return jnp.stack([scale, beta - mean * scale])


def _mol_conv_layer(x, xstats, edges, zeros, p, F, **kw):
    C = F // _CW
    KM = _NBT * _CW
    beg3, end3, bt3 = edges
    xcm = x.reshape(_N, C, _CW).transpose(1, 0, 2).reshape(C * _N, _CW)
    msg = _make_msg_kernel(C)(xcm, beg3, end3, bt3, zeros)
    msg3 = msg.reshape(C, _N, KM)
    mstats = _stats(msg3)

    W, gamma, beta = p["W"], p["gamma"], p["beta"]
    dout = W.shape[0]
    gm = gamma[F:].reshape(_NBT, C, _CW)
    bm = beta[F:].reshape(_NBT, C, _CW)
    Wm = W[:, F:].reshape(dout, _NBT, C, _CW)

    ss_list = [_ss(xstats, gamma[:F], beta[:F])]
    w_list = [W[:, :F].T]
    for c in range(C):
        ss_list.append(_ss(mstats[c], gm[:, c, :].reshape(KM),
                           bm[:, c, :].reshape(KM)))
        w_list.append(Wm[:, :, c, :].reshape(dout, KM).T)
    return _apply([x, msg3], ss_list, w_list, p["b"].reshape(1, dout), dout,
                  **kw)


def _dense_apply(cat, K, stats, p, dout, **kw):
    W, gamma, beta = p["W"], p["gamma"], p["beta"]
    ss = _ss(stats, gamma, beta)
    return _apply([(cat, K)], [ss], [W.T], p["b"].reshape(1, dout), dout, **kw)


def kernel(atom_features, bond_info, params):
    beg = bond_info[:, 0]
    end = bond_info[:, 1]
    bt = bond_info[:, 2]
    pad = _EP - _E
    beg3 = jnp.concatenate(
        [beg, jnp.zeros((pad,), jnp.int32)]).reshape(_NT, _NBATCH, _EB)
    end3 = jnp.concatenate(
        [end, jnp.full((pad,), _N, jnp.int32)]).reshape(_NT, _NBATCH, _EB)
    bt3 = jnp.concatenate(
        [bt, jnp.zeros((pad,), jnp.int32)]).reshape(_NT, _NBATCH, _EB)
    edges = (beg3, end3, bt3)
    zeros = jnp.zeros((_RTL, _CW), jnp.float32)

    x = atom_features
    xstats = _stats(x.reshape(1, _N, 128))[0]
    p0, p1 = params["causal"]
    x, xstats = _mol_conv_layer(x, xstats, edges, zeros, p0, 128)
    cat, hstats = _mol_conv_layer(x, xstats, edges, zeros, p1, 128,
                                  out_width=128 + 4 * 32)

    stats_list = [hstats]
    for li, lp in enumerate(params["dense"]):
        K = 128 + 32 * li
        cstats = jnp.concatenate(stats_list, axis=1)
        bn, bnstats = _dense_apply(cat, K, cstats, lp["bottleneck"], 64)
        cat, kstats = _mol_conv_layer(bn, bnstats, edges, zeros,
                                      lp["conv"], 64,
                                      alias_out=cat, alias_off=32 * li)
        stats_list.append(kstats)

    cstats = jnp.concatenate(stats_list, axis=1)
    out, _ = _dense_apply(cat, 256, cstats, params["output"], 128)
    return out


# speedup vs baseline: 8.2288x; 1.0865x over previous
import functools

import jax
import jax.numpy as jnp
from jax import lax
from jax.experimental import pallas as pl
from jax.experimental.pallas import tpu as pltpu
from jax.experimental.pallas import tpu_sc as plsc

_N = 10000
_E = 320000
_NBT = 4
_N4 = _NBT * _N
_N4P = _N4 + 16
_NT = 16
_EB = 128
_ET = 20224
_NBATCH = _ET // _EB
_NBH = _NBATCH // 2
_EP = _NT * _ET
_RT0 = 2496
_RTL = _N4 - (_NT - 1) * _RT0
_CW = 32
_ROWB = 1000
_NRB = _N // _ROWB


_XR0 = 624
_XRL = _N - (_NT - 1) * _XR0


def _make_msg_kernel(C):
    npc = C // 2
    mesh = plsc.VectorSubcoreMesh(core_axis_name="c", subcore_axis_name="s",
                                  num_cores=2, num_subcores=_NT)

    @functools.partial(
        pl.kernel,
        out_type=jax.ShapeDtypeStruct((C * _N4, _CW), jnp.float32),
        mesh=mesh,
        compiler_params=pltpu.CompilerParams(use_tc_tiling_on_sc=False),
        scratch_types=[
            pltpu.VMEM_SHARED((_N4P, _CW), jnp.float32),
            pltpu.VMEM_SHARED((_N, _CW), jnp.float32),
            pltpu.VMEM((_NBH, _EB), jnp.int32),
            pltpu.VMEM((_NBH, _EB), jnp.int32),
            pltpu.VMEM((2, _EB, _CW), jnp.float32),
            pltpu.SemaphoreType.DMA,
            pltpu.SemaphoreType.DMA,
        ],
    )
    def msg_kernel(x_h, beg_h, end_h, bt_h, zeros_h, out_h,
                   msg_sh, x_sp, beg_v, dst_v, rows_v, sem0, sem1):
        cid = lax.axis_index("c")
        sid = lax.axis_index("s")

        for j in range(npc):
            c = cid * npc + j

            @pl.when(sid < _NT - 1)
            def _():
                pltpu.sync_copy(
                    x_h.at[pl.ds(sid * _XR0, _XR0), pl.ds(c * _CW, _CW)],
                    x_sp.at[pl.ds(sid * _XR0, _XR0)])

            @pl.when(sid == _NT - 1)
            def _():
                pltpu.sync_copy(
                    x_h.at[pl.ds((_NT - 1) * _XR0, _XRL),
                           pl.ds(c * _CW, _CW)],
                    x_sp.at[pl.ds((_NT - 1) * _XR0, _XRL)])

            @pl.when(sid < _NT - 1)
            def _():
                pltpu.sync_copy(zeros_h.at[pl.ds(0, _RT0)],
                                msg_sh.at[pl.ds(sid * _RT0, _RT0)])

            @pl.when(sid == _NT - 1)
            def _():
                pltpu.sync_copy(zeros_h,
                                msg_sh.at[pl.ds((_NT - 1) * _RT0, _RTL)])

            plsc.subcore_barrier()

            for hb in (0, _NBH):
                pltpu.sync_copy(end_h.at[sid].at[pl.ds(hb, _NBH)], dst_v)
                pltpu.sync_copy(bt_h.at[sid].at[pl.ds(hb, _NBH)], beg_v)

                @pl.loop(0, _NBH)
                def _(b):
                    for i in range(_EB // 16):
                        s = pl.ds(i * 16, 16)
                        dst_v[b, s] = dst_v[b, s] * _NBT + beg_v[b, s]

                pltpu.sync_copy(beg_h.at[sid].at[pl.ds(hb, _NBH)], beg_v)

                pltpu.async_copy(x_sp.at[beg_v.at[0]], rows_v.at[0], sem0)

                @pl.loop(0, _NBH // 2)
                def _(k):
                    b0 = k * 2
                    pltpu.make_async_copy(
                        x_sp.at[beg_v.at[b0]], rows_v.at[0], sem0).wait()
                    pltpu.async_copy(
                        x_sp.at[beg_v.at[b0 + 1]], rows_v.at[1], sem1)
                    pltpu.sync_copy(
                        rows_v.at[0], msg_sh.at[dst_v.at[b0]], add=True)
                    pltpu.make_async_copy(
                        x_sp.at[beg_v.at[b0 + 1]], rows_v.at[1], sem1).wait()
                    pltpu.async_copy(
                        x_sp.at[beg_v.at[b0 + 2]], rows_v.at[0], sem0)
                    pltpu.sync_copy(
                        rows_v.at[1], msg_sh.at[dst_v.at[b0 + 1]], add=True)

                bl = _NBH - 1
                pltpu.make_async_copy(
                    x_sp.at[beg_v.at[bl]], rows_v.at[0], sem0).wait()
                pltpu.sync_copy(rows_v.at[0], msg_sh.at[dst_v.at[bl]],
                                add=True)

            plsc.subcore_barrier()

            @pl.when(sid < _NT - 1)
            def _():
                pltpu.sync_copy(
                    msg_sh.at[pl.ds(sid * _RT0, _RT0)],
                    out_h.at[pl.ds(c * _N4 + sid * _RT0, _RT0)])

            @pl.when(sid == _NT - 1)
            def _():
                pltpu.sync_copy(
                    msg_sh.at[pl.ds((_NT - 1) * _RT0, _RTL)],
                    out_h.at[pl.ds(c * _N4 + (_NT - 1) * _RT0, _RTL)])

            if j + 1 < npc:
                plsc.subcore_barrier()

    return msg_kernel


_make_msg_kernel = functools.lru_cache(maxsize=None)(_make_msg_kernel)


def _stats(x3):
    G, _, K = x3.shape

    def body(x_ref, o_ref, acc):
        i = pl.program_id(1)

        @pl.when(i == 0)
        def _():
            acc[...] = jnp.zeros_like(acc)

        xb = x_ref[0]
        acc[0:1, :] += jnp.sum(xb, axis=0, keepdims=True)
        acc[1:2, :] += jnp.sum(xb * xb, axis=0, keepdims=True)

        @pl.when(i == _NRB - 1)
        def _():
            o_ref[0] = acc[...]

    return pl.pallas_call(
        body,
        grid=(G, _NRB),
        in_specs=[pl.BlockSpec((1, _ROWB, K), lambda g, i: (g, i, 0))],
        out_specs=pl.BlockSpec((1, 2, K), lambda g, i: (g, 0, 0)),
        out_shape=jax.ShapeDtypeStruct((G, 2, K), jnp.float32),
        scratch_shapes=[pltpu.VMEM((2, K), jnp.float32)],
    )(x3)


def _apply(blocks, ss_list, w_list, bias, dout, out_width=None,
           alias_out=None, alias_off=0):
    arrs = []
    widths = []
    for b in blocks:
        if isinstance(b, tuple):
            arrs.append(b[0]); widths.append(b[1])
        else:
            arrs.append(b); widths.append(None)
    n_in = len(arrs)
    n_sub = len(w_list)

    def body(*refs):
        bl_refs = refs[:n_in]
        ss_refs = refs[n_in:n_in + n_sub]
        w_refs = refs[n_in + n_sub:n_in + 2 * n_sub]
        bias_ref = refs[n_in + 2 * n_sub]
        k = n_in + 2 * n_sub + 1
        alias_ref = None
        if alias_out is not None:
            alias_ref = refs[k]
            k += 1
        out_ref, st_ref, acc = refs[k:]
        i = pl.program_id(0)

        @pl.when(i == 0)
        def _():
            acc[...] = jnp.zeros_like(acc)

        subs = []
        for r, arr, kw in zip(bl_refs, arrs, widths):
            if arr.ndim == 3:
                for cc in range(arr.shape[0]):
                    subs.append(r[cc])
            elif kw is not None and kw < arr.shape[1]:
                subs.append(r[...][:, :kw])
            else:
                subs.append(r[...])

        tot = None
        for xb, ssr, wr in zip(subs, ss_refs, w_refs):
            s = ssr[...]
            xn = xb * s[0:1, :] + s[1:2, :]
            e = jnp.where(xn > 0, xn, jnp.exp(xn) - 1.0)
            d = jnp.dot(e, wr[...], preferred_element_type=jnp.float32)
            tot = d if tot is None else tot + d
        tot = tot + bias_ref[...]
        if alias_out is None:
            out_ref[...] = tot
        else:
            ob = alias_ref[...]
            parts = []
            if alias_off > 0:
                parts.append(ob[:, :alias_off])
            parts.append(tot)
            if alias_off + dout < ob.shape[1]:
                parts.append(ob[:, alias_off + dout:])
            out_ref[...] = (parts[0] if len(parts) == 1
                            else jnp.concatenate(parts, axis=1))
        acc[0:1, :] += jnp.sum(tot, axis=0, keepdims=True)
        acc[1:2, :] += jnp.sum(tot * tot, axis=0, keepdims=True)

        @pl.when(i == _NRB - 1)
        def _():
            st_ref[...] = acc[...]

    in_specs = []
    for arr, kw in zip(arrs, widths):
        if arr.ndim == 3:
            Cc, _, Kc = arr.shape
            in_specs.append(
                pl.BlockSpec((Cc, _ROWB, Kc), lambda i: (0, i, 0)))
        else:
            in_specs.append(
                pl.BlockSpec((_ROWB, arr.shape[1]), lambda i: (i, 0)))
    for s in ss_list:
        in_specs.append(pl.BlockSpec(s.shape, lambda i: (0, 0)))
    for w in w_list:
        in_specs.append(pl.BlockSpec(w.shape, lambda i: (0, 0)))
    in_specs.append(pl.BlockSpec((1, dout), lambda i: (0, 0)))

    args = list(arrs) + list(ss_list) + list(w_list) + [bias]
    io_aliases = {}
    if alias_out is not None:
        OW = alias_out.shape[1]
        in_specs.append(pl.BlockSpec((_ROWB, 128), lambda i: (i, 1)))
        args.append(alias_out)
        io_aliases = {len(args) - 1: 0}
        out0_shape = jax.ShapeDtypeStruct((_N, OW), jnp.float32)
        out0_spec = pl.BlockSpec((_ROWB, 128), lambda i: (i, 1))
    else:
        OW = out_width if out_width is not None else dout
        out0_shape = jax.ShapeDtypeStruct((_N, OW), jnp.float32)
        out0_spec = pl.BlockSpec((_ROWB, dout), lambda i: (i, 0))

    out, st = pl.pallas_call(
        body,
        grid=(_NRB,),
        in_specs=in_specs,
        out_specs=[
            out0_spec,
            pl.BlockSpec((2, dout), lambda i: (0, 0)),
        ],
        out_shape=[
            out0_shape,
            jax.ShapeDtypeStruct((2, dout), jnp.float32),
        ],
        scratch_shapes=[pltpu.VMEM((2, dout), jnp.float32)],
        input_output_aliases=io_aliases,
    )(*args)
    return out, st


def _ss(stats, gamma, beta):
    mean = stats[0] / _N
    var = stats[1] / _N - mean * mean
    scale = gamma * lax.rsqrt(var + 1e-5)
    return jnp.stack([scale, beta - mean * scale])


def _mol_conv_layer(x, xstats, edges, zeros, p, F, **kw):
    C = F // _CW
    KM = _NBT * _CW
    beg3, end3, bt3 = edges
    msg = _make_msg_kernel(C)(x, beg3, end3, bt3, zeros)
    msg3 = msg.reshape(C, _N, KM)
    mstats = _stats(msg3)

    W, gamma, beta = p["W"], p["gamma"], p["beta"]
    dout = W.shape[0]
    gm = gamma[F:].reshape(_NBT, C, _CW)
    bm = beta[F:].reshape(_NBT, C, _CW)
    Wm = W[:, F:].reshape(dout, _NBT, C, _CW)

    ss_list = [_ss(xstats, gamma[:F], beta[:F])]
    w_list = [W[:, :F].T]
    for c in range(C):
        ss_list.append(_ss(mstats[c], gm[:, c, :].reshape(KM),
                           bm[:, c, :].reshape(KM)))
        w_list.append(Wm[:, :, c, :].reshape(dout, KM).T)
    return _apply([x, msg3], ss_list, w_list, p["b"].reshape(1, dout), dout,
                  **kw)


def _dense_apply(cat, K, stats, p, dout, **kw):
    W, gamma, beta = p["W"], p["gamma"], p["beta"]
    ss = _ss(stats, gamma, beta)
    return _apply([(cat, K)], [ss], [W.T], p["b"].reshape(1, dout), dout, **kw)


def kernel(atom_features, bond_info, params):
    beg = bond_info[:, 0]
    end = bond_info[:, 1]
    bt = bond_info[:, 2]
    pad = _EP - _E
    beg3 = jnp.concatenate(
        [beg, jnp.zeros((pad,), jnp.int32)]).reshape(_NT, _NBATCH, _EB)
    end3 = jnp.concatenate(
        [end, jnp.full((pad,), _N, jnp.int32)]).reshape(_NT, _NBATCH, _EB)
    bt3 = jnp.concatenate(
        [bt, jnp.zeros((pad,), jnp.int32)]).reshape(_NT, _NBATCH, _EB)
    edges = (beg3, end3, bt3)
    zeros = jnp.zeros((_RTL, _CW), jnp.float32)

    x = atom_features
    xstats = _stats(x.reshape(1, _N, 128))[0]
    p0, p1 = params["causal"]
    x, xstats = _mol_conv_layer(x, xstats, edges, zeros, p0, 128)
    cat, hstats = _mol_conv_layer(x, xstats, edges, zeros, p1, 128,
                                  out_width=128 + 4 * 32)

    stats_list = [hstats]
    for li, lp in enumerate(params["dense"]):
        K = 128 + 32 * li
        cstats = jnp.concatenate(stats_list, axis=1)
        bn, bnstats = _dense_apply(cat, K, cstats, lp["bottleneck"], 64)
        cat, kstats = _mol_conv_layer(bn, bnstats, edges, zeros,
                                      lp["conv"], 64,
                                      alias_out=cat, alias_off=32 * li)
        stats_list.append(kstats)

    cstats = jnp.concatenate(stats_list, axis=1)
    out, _ = _dense_apply(cat, 256, cstats, params["output"], 128)
    return out
